# Initial kernel scaffold; baseline (speedup 1.0000x reference)
import jax
import jax.numpy as jnp
from jax.experimental import pallas as pl


def kernel(x, edge_index, W0, b0, Wl, bn_gamma, bn_beta, bn_mean, bn_var, W_out, b_out):
    raise NotImplementedError("write your pallas kernel here")


# speedup vs baseline: 8.8218x; 8.8218x over previous
import functools
import math

import jax
import jax.numpy as jnp
from jax import lax
from jax.experimental import pallas as pl
from jax.experimental.pallas import tpu as pltpu
from jax.experimental.pallas import tpu_sc as plsc

N = 50000
E = 800000
IN = 128
H = 64
OUT = 64
L = 4
ALPHA = 0.1
THETA = 0.5

NC = 2
NS = 16
CHUNK = 128
EDGES_PER_TILE = -(-E // (NS * CHUNK)) * CHUNK
E_PAD = EDGES_PER_TILE * NS
N_CHUNKS = EDGES_PER_TILE // CHUNK

SPLIT = 25024
HALF0 = SPLIT
HALF1 = N - SPLIT
ACC_ROWS = 25088
S0_BIG, S0_LAST = 1568, HALF0 - 15 * 1568
S1_BIG, S1_LAST = 1560, HALF1 - 15 * 1560
ZROWS = ACC_ROWS // NS

ROWBLK = 1000


def _sc_mesh():
    return plsc.VectorSubcoreMesh(core_axis_name="c", subcore_axis_name="s")


def _copy_out(c, s, acc, out_hbm):
    @pl.when(c == 0)
    def _():
        @pl.when(s < NS - 1)
        def _():
            pltpu.sync_copy(acc.at[pl.ds(s * S0_BIG, S0_BIG)],
                            out_hbm.at[pl.ds(s * S0_BIG, S0_BIG)])

        @pl.when(s == NS - 1)
        def _():
            pltpu.sync_copy(acc.at[pl.ds(15 * S0_BIG, S0_LAST)],
                            out_hbm.at[pl.ds(15 * S0_BIG, S0_LAST)])

    @pl.when(c == 1)
    def _():
        @pl.when(s < NS - 1)
        def _():
            pltpu.sync_copy(acc.at[pl.ds(s * S1_BIG, S1_BIG)],
                            out_hbm.at[pl.ds(SPLIT + s * S1_BIG, S1_BIG)])

        @pl.when(s == NS - 1)
        def _():
            pltpu.sync_copy(acc.at[pl.ds(15 * S1_BIG, S1_LAST)],
                            out_hbm.at[pl.ds(SPLIT + 15 * S1_BIG, S1_LAST)])


def _sc_deg_body(col_hbm, ones_hbm, zeros_hbm, deg_hbm, colv, lvv, onesv, acc):
    c = lax.axis_index("c")
    s = lax.axis_index("s")
    pltpu.sync_copy(zeros_hbm, acc.at[pl.ds(s * ZROWS, ZROWS)])
    pltpu.sync_copy(ones_hbm, onesv)
    plsc.subcore_barrier()

    half = jnp.where(c == 0, SPLIT, N - SPLIT)
    base_node = c * SPLIT

    def step(k, carry):
        base = s * EDGES_PER_TILE + k * CHUNK
        pltpu.sync_copy(col_hbm.at[pl.ds(base, CHUNK)], colv)
        for j in range(CHUNK // 16):
            v = colv[pl.ds(j * 16, 16)] - base_node
            ok = (v >= 0) & (v < half)
            lvv[pl.ds(j * 16, 16)] = jnp.where(ok, v, SPLIT)
        pltpu.sync_copy(onesv, acc.at[lvv], add=True)
        return carry

    lax.fori_loop(0, N_CHUNKS, step, 0)
    plsc.subcore_barrier()

    _copy_out(c, s, acc, deg_hbm)


_sc_deg = pl.kernel(
    _sc_deg_body,
    out_type=jax.ShapeDtypeStruct((N, 16), jnp.float32),
    mesh=_sc_mesh(),
    compiler_params=pltpu.CompilerParams(use_tc_tiling_on_sc=False),
    scratch_types=[
        pltpu.VMEM((CHUNK,), jnp.int32),
        pltpu.VMEM((CHUNK,), jnp.int32),
        pltpu.VMEM((CHUNK, 16), jnp.float32),
        pltpu.VMEM_SHARED((ACC_ROWS, 16), jnp.float32),
    ],
)


def _sc_agg_body(hs_hbm, row_hbm, col_hbm, zeros_hbm, s_hbm,
                 colv, rowv, lvv, rows_a, acc, sem_a):
    c = lax.axis_index("c")
    s = lax.axis_index("s")
    for z in range(4):
        pltpu.sync_copy(zeros_hbm,
                        acc.at[pl.ds(s * ZROWS + z * (ZROWS // 4), ZROWS // 4)])
    plsc.subcore_barrier()

    half = jnp.where(c == 0, SPLIT, N - SPLIT)
    base_node = c * SPLIT
    tile_base = s * EDGES_PER_TILE

    def step(k, carry):
        base = tile_base + k * CHUNK
        pltpu.sync_copy(row_hbm.at[pl.ds(base, CHUNK)], rowv)
        d = pltpu.async_copy(hs_hbm.at[rowv], rows_a, sem_a)
        pltpu.sync_copy(col_hbm.at[pl.ds(base, CHUNK)], colv)
        for j in range(CHUNK // 16):
            v = colv[pl.ds(j * 16, 16)] - base_node
            ok = (v >= 0) & (v < half)
            lvv[pl.ds(j * 16, 16)] = jnp.where(ok, v, SPLIT)
        d.wait()
        pltpu.sync_copy(rows_a, acc.at[lvv], add=True)
        return carry

    lax.fori_loop(0, N_CHUNKS, step, 0)
    plsc.subcore_barrier()

    _copy_out(c, s, acc, s_hbm)


_sc_agg = pl.kernel(
    _sc_agg_body,
    out_type=jax.ShapeDtypeStruct((N, H), jnp.float32),
    mesh=_sc_mesh(),
    compiler_params=pltpu.CompilerParams(use_tc_tiling_on_sc=False),
    scratch_types=[
        pltpu.VMEM((CHUNK,), jnp.int32),
        pltpu.VMEM((CHUNK,), jnp.int32),
        pltpu.VMEM((CHUNK,), jnp.int32),
        pltpu.VMEM((CHUNK, H), jnp.float32),
        pltpu.VMEM_SHARED((ACC_ROWS, H), jnp.float32),
        pltpu.SemaphoreType.DMA,
    ],
)


def _tc_in_body(x_ref, w_ref, b_ref, deg_ref, h_ref, hs_ref):
    h = jnp.dot(x_ref[...], w_ref[...],
                preferred_element_type=jnp.float32,
                precision=lax.Precision.HIGHEST)
    h = jnp.maximum(h + b_ref[...], 0.0)
    dinv = lax.rsqrt(deg_ref[:, 0:1] + 1.0)
    h_ref[...] = h
    hs_ref[...] = h * dinv


def _tc_in(x, w0, b0, deg):
    grid = (N // ROWBLK,)
    return pl.pallas_call(
        _tc_in_body,
        grid=grid,
        in_specs=[
            pl.BlockSpec((ROWBLK, IN), lambda i: (i, 0)),
            pl.BlockSpec((IN, H), lambda i: (0, 0)),
            pl.BlockSpec((1, H), lambda i: (0, 0)),
            pl.BlockSpec((ROWBLK, 16), lambda i: (i, 0)),
        ],
        out_specs=[
            pl.BlockSpec((ROWBLK, H), lambda i: (i, 0)),
            pl.BlockSpec((ROWBLK, H), lambda i: (i, 0)),
        ],
        out_shape=[
            jax.ShapeDtypeStruct((N, H), jnp.float32),
            jax.ShapeDtypeStruct((N, H), jnp.float32),
        ],
    )(x, w0, b0.reshape(1, H), deg)


def _tc_layer_body(beta, s_ref, hs_ref, x0_ref, deg_ref, w_ref,
                   gam_ref, bet_ref, mu_ref, var_ref, h_ref, hsn_ref):
    dinv = lax.rsqrt(deg_ref[:, 0:1] + 1.0)
    agg = dinv * (s_ref[...] + hs_ref[...])
    out = (1.0 - ALPHA) * agg + ALPHA * x0_ref[...]
    t = (1.0 - beta) * out + beta * jnp.dot(
        out, w_ref[...], preferred_element_type=jnp.float32,
        precision=lax.Precision.HIGHEST)
    scale = gam_ref[...] * lax.rsqrt(var_ref[...] + 1e-5)
    t = (t - mu_ref[...]) * scale + bet_ref[...]
    h = jnp.maximum(t, 0.0)
    h_ref[...] = h
    hsn_ref[...] = h * dinv


def _tc_layer(beta, s, hs, x0, deg, w, gam, bet, mu, var):
    grid = (N // ROWBLK,)
    rb = lambda i: (i, 0)
    z = lambda i: (0, 0)
    return pl.pallas_call(
        functools.partial(_tc_layer_body, beta),
        grid=grid,
        in_specs=[
            pl.BlockSpec((ROWBLK, H), rb),
            pl.BlockSpec((ROWBLK, H), rb),
            pl.BlockSpec((ROWBLK, H), rb),
            pl.BlockSpec((ROWBLK, 16), rb),
            pl.BlockSpec((H, H), z),
            pl.BlockSpec((1, H), z),
            pl.BlockSpec((1, H), z),
            pl.BlockSpec((1, H), z),
            pl.BlockSpec((1, H), z),
        ],
        out_specs=[
            pl.BlockSpec((ROWBLK, H), rb),
            pl.BlockSpec((ROWBLK, H), rb),
        ],
        out_shape=[
            jax.ShapeDtypeStruct((N, H), jnp.float32),
            jax.ShapeDtypeStruct((N, H), jnp.float32),
        ],
    )(s, hs, x0, deg, w, gam.reshape(1, H), bet.reshape(1, H),
      mu.reshape(1, H), var.reshape(1, H))


def _tc_out_body(h_ref, w_ref, b_ref, y_ref):
    y_ref[...] = jnp.dot(h_ref[...], w_ref[...],
                         preferred_element_type=jnp.float32,
                         precision=lax.Precision.HIGHEST) + b_ref[...]


def _tc_out(h, w, b):
    grid = (N // ROWBLK,)
    return pl.pallas_call(
        _tc_out_body,
        grid=grid,
        in_specs=[
            pl.BlockSpec((ROWBLK, H), lambda i: (i, 0)),
            pl.BlockSpec((H, OUT), lambda i: (0, 0)),
            pl.BlockSpec((1, OUT), lambda i: (0, 0)),
        ],
        out_specs=pl.BlockSpec((ROWBLK, OUT), lambda i: (i, 0)),
        out_shape=jax.ShapeDtypeStruct((N, OUT), jnp.float32),
    )(h, w, b.reshape(1, OUT))


def kernel(x, edge_index, W0, b0, Wl, bn_gamma, bn_beta, bn_mean, bn_var,
           W_out, b_out):
    row = edge_index[0]
    col = edge_index[1]
    row_pad = jnp.concatenate(
        [row, jnp.zeros((E_PAD - E,), jnp.int32)])
    col_pad = jnp.concatenate(
        [col, jnp.full((E_PAD - E,), -1, jnp.int32)])

    ones16 = jnp.ones((CHUNK, 16), jnp.float32)
    zeros16 = jnp.zeros((ZROWS, 16), jnp.float32)
    zeros64 = jnp.zeros((ZROWS // 4, H), jnp.float32)

    deg = _sc_deg(col_pad, ones16, zeros16)
    h, hs = _tc_in(x, W0, b0, deg)
    x0 = h

    for i in range(L):
        beta = float(math.log(THETA / (i + 1) + 1.0))
        s = _sc_agg(hs, row_pad, col_pad, zeros64)
        h, hs = _tc_layer(beta, s, hs, x0, deg, Wl[i], bn_gamma[i],
                          bn_beta[i], bn_mean[i], bn_var[i])

    return _tc_out(h, W_out, b_out)
